# baseline (speedup 1.0000x reference)
import functools
import math

import jax
import jax.numpy as jnp
import numpy as np
from jax import lax
from jax.experimental import pallas as pl
from jax.experimental.pallas import tpu as pltpu
from jax.experimental.pallas import tpu_sc as plsc

_EMB = 128
_SEQ = 200
_BATCH = 1024
_D_MODEL = 128
_SCALE = math.sqrt(float(_D_MODEL))

_NC = 2
_NS = 16
_NW = _NC * _NS
_PB = 16
_PS = _NW // _PB
_BS = _BATCH // _PB
_SS = _SEQ // _PS
_NBUF = 6
_DEPTH = 4
_LPR = _EMB // 16


def _pos_emb():
    pos = jnp.arange(_SEQ, dtype=jnp.float32)[:, None]
    freq = jnp.exp(
        jnp.arange(0, _D_MODEL, 2, dtype=jnp.float32)
        * -(np.log(10000.0) / _D_MODEL)
    )[None, :]
    args = pos * freq
    emb = jnp.zeros((_SEQ, _EMB), dtype=jnp.float32)
    emb = emb.at[:, 0::2].set(jnp.sin(args))
    emb = emb.at[:, 1::2].set(jnp.cos(args))
    return emb


_mesh = plsc.VectorSubcoreMesh(core_axis_name="c", subcore_axis_name="s")


@functools.partial(
    pl.kernel,
    mesh=_mesh,
    compiler_params=pltpu.CompilerParams(use_tc_tiling_on_sc=False),
    out_type=jax.ShapeDtypeStruct((_BATCH, _SEQ, _EMB), jnp.float32),
    scratch_types=[
        pltpu.VMEM((_SS, _BS), jnp.int32),
        pltpu.VMEM((_SS, _EMB), jnp.float32),
    ]
    + [pltpu.VMEM((_BS, _EMB), jnp.float32) for _ in range(_NBUF)]
    + [pltpu.SemaphoreType.DMA for _ in range(2 * _NBUF)],
)
def _emb_kernel(xt_hbm, table_hbm, pos_hbm, out_hbm, idx_v, pos_v, *bufs_and_sems):
    rows = bufs_and_sems[:_NBUF]
    g_sem = bufs_and_sems[_NBUF:2 * _NBUF]
    s_sem = bufs_and_sems[2 * _NBUF:]

    wid = lax.axis_index("s") * _NC + lax.axis_index("c")
    wb = lax.rem(wid, _PB)
    ws = wid // _PB
    b0 = wb * _BS
    s0 = ws * _SS
    idx_cp = pltpu.async_copy(
        xt_hbm.at[pl.ds(s0, _SS), pl.ds(b0, _BS)], idx_v, g_sem[_NBUF - 1]
    )
    pos_cp = pltpu.async_copy(pos_hbm.at[pl.ds(s0, _SS)], pos_v, s_sem[_NBUF - 1])
    idx_cp.wait()

    gathers = {}
    scatters = {}

    def issue_gather(i):
        b = i % _NBUF
        gathers[i] = pltpu.async_copy(
            table_hbm.at[idx_v.at[i]], rows[b], g_sem[b]
        )

    def compute(b, i):
        buf = rows[b]
        pv = [pos_v[i, pl.ds(c * 16, 16)] for c in range(_LPR)]

        def row_body(r, c2):
            for c in range(_LPR):
                sl = pl.ds(c * 16, 16)
                buf[r, sl] = buf[r, sl] * _SCALE + pv[c]
            return c2

        lax.fori_loop(0, _BS, row_body, 0)

    for j in range(_DEPTH):
        issue_gather(j)
    pos_cp.wait()

    for i in range(_SS):
        b = i % _NBUF
        gathers[i].wait()
        compute(b, i)
        scatters[i] = pltpu.async_copy(
            rows[b], out_hbm.at[pl.ds(b0, _BS), s0 + i], s_sem[b]
        )
        j = i + _DEPTH
        if j < _SS:
            if j >= _NBUF:
                scatters[j - _NBUF].wait()
            issue_gather(j)

    for i in range(_SS - _NBUF, _SS):
        scatters[i].wait()


def kernel(x, table):
    xt = x.astype(jnp.int32).T
    return _emb_kernel(xt, table, _pos_emb())


# speedup vs baseline: 1.0488x; 1.0488x over previous
import functools
import math

import jax
import jax.numpy as jnp
import numpy as np
from jax import lax
from jax.experimental import pallas as pl
from jax.experimental.pallas import tpu as pltpu
from jax.experimental.pallas import tpu_sc as plsc

_EMB = 128
_SEQ = 200
_BATCH = 1024
_D_MODEL = 128
_SCALE = math.sqrt(float(_D_MODEL))

_NC = 2
_NS = 16
_NW = _NC * _NS
_PB = 8
_PS = _NW // _PB
_BS = _BATCH // _PB
_SS = _SEQ // _PS
_NBUF = 6
_DEPTH = 4
_LPR = _EMB // 16


def _pos_emb():
    pos = jnp.arange(_SEQ, dtype=jnp.float32)[:, None]
    freq = jnp.exp(
        jnp.arange(0, _D_MODEL, 2, dtype=jnp.float32)
        * -(np.log(10000.0) / _D_MODEL)
    )[None, :]
    args = pos * freq
    emb = jnp.zeros((_SEQ, _EMB), dtype=jnp.float32)
    emb = emb.at[:, 0::2].set(jnp.sin(args))
    emb = emb.at[:, 1::2].set(jnp.cos(args))
    return emb


_mesh = plsc.VectorSubcoreMesh(core_axis_name="c", subcore_axis_name="s")


@functools.partial(
    pl.kernel,
    mesh=_mesh,
    compiler_params=pltpu.CompilerParams(use_tc_tiling_on_sc=False),
    out_type=jax.ShapeDtypeStruct((_BATCH, _SEQ, _EMB), jnp.float32),
    scratch_types=[
        pltpu.VMEM((_SS, _BS), jnp.int32),
        pltpu.VMEM((_SS, _EMB), jnp.float32),
    ]
    + [pltpu.VMEM((_BS, _EMB), jnp.float32) for _ in range(_NBUF)]
    + [pltpu.SemaphoreType.DMA for _ in range(2 * _NBUF)],
)
def _emb_kernel(xt_hbm, table_hbm, pos_hbm, out_hbm, idx_v, pos_v, *bufs_and_sems):
    rows = bufs_and_sems[:_NBUF]
    g_sem = bufs_and_sems[_NBUF:2 * _NBUF]
    s_sem = bufs_and_sems[2 * _NBUF:]

    wid = lax.axis_index("s") * _NC + lax.axis_index("c")
    wb = lax.rem(wid, _PB)
    ws = wid // _PB
    b0 = wb * _BS
    s0 = ws * _SS
    idx_cp1 = pltpu.async_copy(
        xt_hbm.at[pl.ds(s0, _DEPTH), pl.ds(b0, _BS)],
        idx_v.at[pl.ds(0, _DEPTH)], g_sem[_NBUF - 1],
    )
    pos_cp = pltpu.async_copy(pos_hbm.at[pl.ds(s0, _SS)], pos_v, s_sem[_NBUF - 1])
    idx_cp1.wait()
    idx_cp2 = pltpu.async_copy(
        xt_hbm.at[pl.ds(s0 + _DEPTH, _SS - _DEPTH), pl.ds(b0, _BS)],
        idx_v.at[pl.ds(_DEPTH, _SS - _DEPTH)], g_sem[_NBUF - 1],
    )

    gathers = {}
    scatters = {}

    def issue_gather(i):
        b = i % _NBUF
        gathers[i] = pltpu.async_copy(
            table_hbm.at[idx_v.at[i]], rows[b], g_sem[b]
        )

    def compute(b, i):
        buf = rows[b]
        pv = [pos_v[i, pl.ds(c * 16, 16)] for c in range(_LPR)]

        def row_body(r, c2):
            for c in range(_LPR):
                sl = pl.ds(c * 16, 16)
                buf[r, sl] = buf[r, sl] * _SCALE + pv[c]
            return c2

        lax.fori_loop(0, _BS, row_body, 0)

    for j in range(_DEPTH):
        issue_gather(j)
    idx_cp2.wait()
    pos_cp.wait()

    for i in range(_SS):
        b = i % _NBUF
        gathers[i].wait()
        compute(b, i)
        scatters[i] = pltpu.async_copy(
            rows[b], out_hbm.at[pl.ds(b0, _BS), s0 + i], s_sem[b]
        )
        j = i + _DEPTH
        if j < _SS:
            if j >= _NBUF:
                scatters[j - _NBUF].wait()
            issue_gather(j)

    for i in range(_SS - _NBUF, _SS):
        scatters[i].wait()


def kernel(x, table):
    xt = x.astype(jnp.int32).T
    return _emb_kernel(xt, table, _pos_emb())
